# Initial kernel scaffold; baseline (speedup 1.0000x reference)
import jax
import jax.numpy as jnp
from jax.experimental import pallas as pl


def kernel(h_op, h_mac, seq_src, seq_dst, op_mac_src, op_mac_dst, mac_op_src, mac_op_dst, feat_seq, feat_op_mac, feat_mac_op, W_op_w, W_op_b, W_mac_w, W_mac_b, att_seq, att_op_mac, att_mac_op, ln_op_s, ln_op_b, ln_mac_s, ln_mac_b):
    raise NotImplementedError("write your pallas kernel here")


# speedup vs baseline: 10.1733x; 10.1733x over previous
import functools
import jax
import jax.numpy as jnp
from jax.experimental import pallas as pl

N_OP_C = 100000
N_MAC_C = 10000
D = 128
HEADS_C = 4
DK_C = 32
EPS_C = 1e-06
LN_EPS_C = 1e-05


def _lin_proj_body(h_ref, wt_ref, b_ref, a_ref, lin_ref, sc_ref):
    h = h_ref[...]
    lin = jnp.dot(h, wt_ref[...], preferred_element_type=jnp.float32) + b_ref[...]
    lin_ref[...] = lin
    sc_ref[...] = jnp.dot(lin, a_ref[...], preferred_element_type=jnp.float32)


def _lin_proj(h, wt, b, a, blk):
    n = h.shape[0]
    grid = (n + blk - 1) // blk
    return pl.pallas_call(
        _lin_proj_body,
        grid=(grid,),
        in_specs=[
            pl.BlockSpec((blk, D), lambda i: (i, 0)),
            pl.BlockSpec((D, D), lambda i: (0, 0)),
            pl.BlockSpec((1, D), lambda i: (0, 0)),
            pl.BlockSpec((D, 16), lambda i: (0, 0)),
        ],
        out_specs=[
            pl.BlockSpec((blk, D), lambda i: (i, 0)),
            pl.BlockSpec((blk, 16), lambda i: (i, 0)),
        ],
        out_shape=[
            jax.ShapeDtypeStruct((n, D), jnp.float32),
            jax.ShapeDtypeStruct((n, 16), jnp.float32),
        ],
    )(h, wt, b, a)


def _epilogue_body(out_ref, lin_ref, s_ref, b_ref, res_ref):
    x = out_ref[...] + lin_ref[...]
    mu = jnp.mean(x, axis=-1, keepdims=True)
    xc = x - mu
    var = jnp.mean(xc * xc, axis=-1, keepdims=True)
    y = xc * jax.lax.rsqrt(var + LN_EPS_C) * s_ref[...] + b_ref[...]
    res_ref[...] = jnp.where(y > 0, y, jnp.exp(jnp.minimum(y, 0.0)) - 1.0)


def _epilogue(out, lin, s, b, blk):
    n = out.shape[0]
    grid = (n + blk - 1) // blk
    return pl.pallas_call(
        _epilogue_body,
        grid=(grid,),
        in_specs=[
            pl.BlockSpec((blk, D), lambda i: (i, 0)),
            pl.BlockSpec((blk, D), lambda i: (i, 0)),
            pl.BlockSpec((1, D), lambda i: (0, 0)),
            pl.BlockSpec((1, D), lambda i: (0, 0)),
        ],
        out_specs=pl.BlockSpec((blk, D), lambda i: (i, 0)),
        out_shape=jax.ShapeDtypeStruct((n, D), jnp.float32),
    )(out, lin, s, b)


def _build_proj_mats(att_seq, att_op_mac, att_mac_op):
    def col(att, h, half):
        c = jnp.zeros((D,), jnp.float32)
        seg = att[h, half * DK_C:(half + 1) * DK_C]
        return c.at[h * DK_C:(h + 1) * DK_C].set(seg)

    cols_op = []
    cols_mac = []
    for att, half in [(att_seq, 0), (att_seq, 1), (att_op_mac, 0), (att_mac_op, 1)]:
        for h in range(HEADS_C):
            cols_op.append(col(att, h, half))
    for att, half in [(att_op_mac, 1), (att_mac_op, 0)]:
        for h in range(HEADS_C):
            cols_mac.append(col(att, h, half))
    for _ in range(8):
        cols_mac.append(jnp.zeros((D,), jnp.float32))
    return jnp.stack(cols_op, axis=1), jnp.stack(cols_mac, axis=1)


def _edge_phase(sc_src, sc_dst, z_src, src, dst, feat, attf, cs, cd, n_dst):
    ss = sc_src[src, cs:cs + HEADS_C]
    sd = sc_dst[dst, cd:cd + HEADS_C]
    scores = ss + sd + feat * attf[None, :]
    scores = jnp.where(scores >= 0, scores, 0.2 * scores)
    scores = jnp.clip(scores, -20.0, 20.0)
    alpha = jnp.exp(scores)
    denom = jnp.zeros((n_dst, HEADS_C), jnp.float32).at[dst].add(alpha)
    norm = alpha / (denom[dst] + EPS_C)
    weighted = z_src[src] * jnp.repeat(norm, DK_C, axis=1)
    return jnp.zeros((n_dst, D), jnp.float32).at[dst].add(weighted)


def kernel(h_op, h_mac, seq_src, seq_dst, op_mac_src, op_mac_dst,
           mac_op_src, mac_op_dst, feat_seq, feat_op_mac, feat_mac_op,
           W_op_w, W_op_b, W_mac_w, W_mac_b,
           att_seq, att_op_mac, att_mac_op,
           ln_op_s, ln_op_b, ln_mac_s, ln_mac_b):
    a_op, a_mac = _build_proj_mats(att_seq, att_op_mac, att_mac_op)
    lin_op, sc_op = _lin_proj(h_op, W_op_w.T, W_op_b[None, :], a_op, 2000)
    lin_mac, sc_mac = _lin_proj(h_mac, W_mac_w.T, W_mac_b[None, :], a_mac, 2000)

    out_op = _edge_phase(sc_op, sc_op, lin_op, seq_src, seq_dst,
                         feat_seq, att_seq[:, 2 * DK_C], 0, 4, N_OP_C)
    out_op = out_op + _edge_phase(sc_mac, sc_op, lin_mac, mac_op_src, mac_op_dst,
                                  feat_mac_op, att_mac_op[:, 2 * DK_C], 4, 12, N_OP_C)
    out_mac = _edge_phase(sc_op, sc_mac, lin_op, op_mac_src, op_mac_dst,
                          feat_op_mac, att_op_mac[:, 2 * DK_C], 8, 0, N_MAC_C)

    res_op = _epilogue(out_op, lin_op, ln_op_s[None, :], ln_op_b[None, :], 2000)
    res_mac = _epilogue(out_mac, lin_mac, ln_mac_s[None, :], ln_mac_b[None, :], 2000)
    return (res_op, res_mac)
